# baseline (speedup 1.0000x reference)
import functools

import jax
import jax.numpy as jnp
from jax import lax
from jax.experimental import pallas as pl
from jax.experimental.pallas import tpu as pltpu
from jax.experimental.pallas import tpu_sc as plsc

_T, _H, _E, _K = 2048, 1024, 8, 2
_F, _FS = 768, 2048
_M = 128
_NT = (_K * _T) // _M + _E
_NTM = _NT * _M
_MT = 256
_NW = 32
_CH = 32


def _router_body(x_ref, rw_ref, idx_ref, w_ref):
    x = x_ref[...]
    logits = lax.dot_general(x, rw_ref[...], (((1,), (1,)), ((), ())),
                             preferred_element_type=jnp.float32)
    lane = lax.broadcasted_iota(jnp.int32, (_T, _E), 1)
    m = jnp.max(logits, axis=1, keepdims=True)
    ex = jnp.exp(logits - m)
    probs = ex / jnp.sum(ex, axis=1, keepdims=True)
    big = jnp.int32(_E)
    m0 = jnp.max(probs, axis=1, keepdims=True)
    i0 = jnp.min(jnp.where(probs == m0, lane, big), axis=1, keepdims=True)
    probs1 = jnp.where(lane == i0, -1.0, probs)
    m1 = jnp.max(probs1, axis=1, keepdims=True)
    i1 = jnp.min(jnp.where(probs1 == m1, lane, big), axis=1, keepdims=True)
    s = m0 + m1
    idx_ref[...] = jnp.concatenate([i0, i1], axis=1)
    w_ref[...] = jnp.concatenate([m0 / s, m1 / s], axis=1)


def _expert_body(te_ref, x_ref, g_ref, u_ref, d_ref, o_ref):
    x = x_ref[...]
    g = lax.dot_general(x, g_ref[0], (((1,), (1,)), ((), ())),
                        preferred_element_type=jnp.float32)
    u = lax.dot_general(x, u_ref[0], (((1,), (1,)), ((), ())),
                        preferred_element_type=jnp.float32)
    a = g * lax.logistic(g) * u
    o_ref[...] = lax.dot_general(a, d_ref[0], (((1,), (1,)), ((), ())),
                                 preferred_element_type=jnp.float32)


def _shared_body(x_ref, sg_ref, su_ref, sd_ref, gp_ref, rg_ref, tw_ref, o_ref):
    x = x_ref[...]
    g = lax.dot_general(x, sg_ref[...], (((1,), (1,)), ((), ())),
                        preferred_element_type=jnp.float32)
    u = lax.dot_general(x, su_ref[...], (((1,), (1,)), ((), ())),
                        preferred_element_type=jnp.float32)
    a = g * lax.logistic(g) * u
    sh = lax.dot_general(a, sd_ref[...], (((1,), (1,)), ((), ())),
                         preferred_element_type=jnp.float32)
    gate = lax.logistic(
        lax.dot_general(x, gp_ref[...], (((1,), (1,)), ((), ())),
                        preferred_element_type=jnp.float32))
    rg = rg_ref[...]
    tw = tw_ref[...]
    o_ref[...] = (gate * sh
                  + tw[:, 0:1] * rg[:, :_H]
                  + tw[:, 1:2] * rg[:, _H:])


def _gather_rows(table, idx, nch):
    d = table.shape[1]
    b = _NW * nch * _CH
    idx3 = idx.reshape(_NW, nch, _CH)
    mesh = plsc.VectorSubcoreMesh(core_axis_name="c", subcore_axis_name="s")

    @functools.partial(
        pl.kernel, mesh=mesh,
        out_type=jax.ShapeDtypeStruct((b, d), jnp.float32),
        scratch_types=[
            pltpu.VMEM((nch, _CH), jnp.int32),
            pltpu.VMEM((_CH, d), jnp.float32),
            pltpu.SemaphoreType.DMA,
        ],
    )
    def k(table_hbm, idx_hbm, out_hbm, idx_v, rows_v, sem):
        wid = lax.axis_index("s") * 2 + lax.axis_index("c")
        base = wid * (nch * _CH)
        pltpu.sync_copy(idx_hbm.at[wid], idx_v)
        for c in range(nch):
            pltpu.async_copy(table_hbm.at[idx_v.at[c]], rows_v, sem).wait()
            pltpu.sync_copy(rows_v, out_hbm.at[pl.ds(base + c * _CH, _CH)])

    return k(table, idx3)


def kernel(hidden_states, router_w, gate_w, up_w, down_w, sg_w, su_w, sd_w,
           shared_gate_w):
    hs = hidden_states.reshape(_T, _H)

    top_i, top_w = pl.pallas_call(
        _router_body,
        out_shape=(jax.ShapeDtypeStruct((_T, _K), jnp.int32),
                   jax.ShapeDtypeStruct((_T, _K), jnp.float32)),
    )(hs, router_w)

    e_flat = jnp.concatenate([top_i[:, 0], top_i[:, 1]])
    onehot = (e_flat[:, None] == jnp.arange(_E)[None, :]).astype(jnp.int32)
    ranks_excl = jnp.cumsum(onehot, axis=0) - onehot
    rank = jnp.take_along_axis(ranks_excl, e_flat[:, None], axis=1)[:, 0]
    counts = jnp.sum(onehot, axis=0)
    tiles_per_e = (counts + _M - 1) // _M
    tile_off = jnp.concatenate([jnp.zeros(1, jnp.int32),
                                jnp.cumsum(tiles_per_e)[:-1].astype(jnp.int32)])
    pos = tile_off[e_flat] * _M + rank
    tok = jnp.concatenate([jnp.arange(_T, dtype=jnp.int32)] * 2)
    src = jnp.zeros(_NTM, jnp.int32).at[pos].set(tok)
    ends = tile_off + tiles_per_e
    tile_expert = jnp.minimum(
        jnp.sum(jnp.arange(_NT)[:, None] >= ends[None, :], axis=1), _E - 1
    ).astype(jnp.int32)
    pos_k = pos.reshape(2, _T)
    qf = jnp.stack([pos_k[0], pos_k[1]], axis=1).reshape(-1)

    x_sorted = _gather_rows(hs, src, _NTM // (_NW * _CH))

    grid_spec = pltpu.PrefetchScalarGridSpec(
        num_scalar_prefetch=1,
        grid=(_NT,),
        in_specs=[
            pl.BlockSpec((_M, _H), lambda i, te: (i, 0)),
            pl.BlockSpec((1, _F, _H), lambda i, te: (te[i], 0, 0)),
            pl.BlockSpec((1, _F, _H), lambda i, te: (te[i], 0, 0)),
            pl.BlockSpec((1, _H, _F), lambda i, te: (te[i], 0, 0)),
        ],
        out_specs=pl.BlockSpec((_M, _H), lambda i, te: (i, 0)),
    )
    contrib = pl.pallas_call(
        _expert_body,
        grid_spec=grid_spec,
        out_shape=jax.ShapeDtypeStruct((_NTM, _H), jnp.float32),
        compiler_params=pltpu.CompilerParams(
            dimension_semantics=("arbitrary",)),
    )(tile_expert, x_sorted, gate_w, up_w, down_w)

    rows_g = _gather_rows(contrib, qf, (2 * _T) // (_NW * _CH))
    rg = rows_g.reshape(_T, 2 * _H)

    out = pl.pallas_call(
        _shared_body,
        grid=(_T // _MT,),
        in_specs=[
            pl.BlockSpec((_MT, _H), lambda i: (i, 0)),
            pl.BlockSpec((_FS, _H), lambda i: (0, 0)),
            pl.BlockSpec((_FS, _H), lambda i: (0, 0)),
            pl.BlockSpec((_H, _FS), lambda i: (0, 0)),
            pl.BlockSpec((1, _H), lambda i: (0, 0)),
            pl.BlockSpec((_MT, 2 * _H), lambda i: (i, 0)),
            pl.BlockSpec((_MT, _K), lambda i: (i, 0)),
        ],
        out_specs=pl.BlockSpec((_MT, _H), lambda i: (i, 0)),
        out_shape=jax.ShapeDtypeStruct((_T, _H), jnp.float32),
    )(hs, sg_w, su_w, sd_w, shared_gate_w, rg, top_w)

    return out.reshape(1, _T, _H)
